# baseline (speedup 1.0000x reference)
import jax
import jax.numpy as jnp
from jax.experimental import pallas as pl
from jax.experimental.pallas import tpu as pltpu

_VBS = 1024
_EPS = 1e-5
_N_BISECT = 11
_N_NEWTON = 2


def _fused_chunk(a_ref, p_ref, w_ref, b_ref, g_ref, bt_ref, o_ref):
    d = jnp.float32(o_ref.shape[1])
    h = jax.lax.dot_general(
        a_ref[...], w_ref[...], (((1,), (1,)), ((), ())),
        preferred_element_type=jnp.float32,
    )
    h = h + b_ref[...]
    outs = []
    for j in range(0, h.shape[0], _VBS):
        hc = h[j:j + _VBS]
        mean = jnp.mean(hc, axis=0, keepdims=True)
        var = jnp.mean(jnp.square(hc - mean), axis=0, keepdims=True)
        outs.append((hc - mean) * jax.lax.rsqrt(var + _EPS))
    hn = outs[0] if len(outs) == 1 else jnp.concatenate(outs, axis=0)
    z = (hn * g_ref[...] + bt_ref[...]) * p_ref[...]
    zt = z.T
    m = jnp.max(zt, axis=0, keepdims=True)
    lo = m - 1.0
    for i in range(1, _N_BISECT + 1):
        cand = lo + jnp.float32(2.0 ** -i)
        fs = jnp.sum(jnp.maximum(zt, cand), axis=0, keepdims=True)
        lo = jnp.where(fs - d * cand > 1.0, cand, lo)
    tau = lo
    for _ in range(_N_NEWTON):
        fs = jnp.sum(jnp.maximum(zt, tau), axis=0, keepdims=True)
        k = jnp.sum((zt > tau).astype(jnp.float32), axis=0, keepdims=True)
        tau = tau + (fs - d * tau - 1.0) / k
    o_ref[...] = jnp.maximum(z - tau.T, 0.0)


def kernel(a, priors, W, b, gamma, beta):
    n, d_a = a.shape
    inp_dim = W.shape[0]
    block = min(n, 4 * _VBS)
    b2 = b.reshape(1, inp_dim)
    g2 = gamma.reshape(1, inp_dim)
    bt2 = beta.reshape(1, inp_dim)
    return pl.pallas_call(
        _fused_chunk,
        grid=(n // block,),
        in_specs=[
            pl.BlockSpec((block, d_a), lambda i: (i, 0)),
            pl.BlockSpec((block, inp_dim), lambda i: (i, 0)),
            pl.BlockSpec((inp_dim, d_a), lambda i: (0, 0)),
            pl.BlockSpec((1, inp_dim), lambda i: (0, 0)),
            pl.BlockSpec((1, inp_dim), lambda i: (0, 0)),
            pl.BlockSpec((1, inp_dim), lambda i: (0, 0)),
        ],
        out_specs=pl.BlockSpec((block, inp_dim), lambda i: (i, 0)),
        out_shape=jax.ShapeDtypeStruct((n, inp_dim), jnp.float32),
        compiler_params=pltpu.CompilerParams(
            dimension_semantics=("parallel",),
        ),
    )(a, priors, W, b2, g2, bt2)


# speedup vs baseline: 1.0568x; 1.0568x over previous
import jax
import jax.numpy as jnp
from jax.experimental import pallas as pl
from jax.experimental.pallas import tpu as pltpu

_VBS = 1024
_EPS = 1e-5
_N_BISECT = 11
_N_NEWTON = 2


def _fused_chunk(a_ref, p_ref, wt_ref, b_ref, g_ref, bt_ref, o_ref):
    d = jnp.float32(o_ref.shape[1])
    h = jnp.dot(a_ref[...], wt_ref[...], preferred_element_type=jnp.float32)
    h = h + b_ref[...]
    outs = []
    for j in range(0, h.shape[0], _VBS):
        hc = h[j:j + _VBS]
        mean = jnp.mean(hc, axis=0, keepdims=True)
        var = jnp.mean(jnp.square(hc - mean), axis=0, keepdims=True)
        outs.append((hc - mean) * jax.lax.rsqrt(var + _EPS))
    hn = outs[0] if len(outs) == 1 else jnp.concatenate(outs, axis=0)
    z = (hn * g_ref[...] + bt_ref[...]) * p_ref[...]
    zt = z.T
    m = jnp.max(zt, axis=0, keepdims=True)
    lo = m - 1.0
    for i in range(1, _N_BISECT + 1):
        cand = lo + jnp.float32(2.0 ** -i)
        fs = jnp.sum(jnp.maximum(zt, cand), axis=0, keepdims=True)
        lo = jnp.where(fs - d * cand > 1.0, cand, lo)
    tau = lo
    for _ in range(_N_NEWTON):
        fs = jnp.sum(jnp.maximum(zt, tau), axis=0, keepdims=True)
        k = jnp.sum((zt > tau).astype(jnp.float32), axis=0, keepdims=True)
        tau = tau + (fs - d * tau - 1.0) / k
    o_ref[...] = jnp.maximum(z - tau.T, 0.0)


def kernel(a, priors, W, b, gamma, beta):
    n, d_a = a.shape
    inp_dim = W.shape[0]
    block = min(n, 4 * _VBS)
    wt = W.T
    b2 = b.reshape(1, inp_dim)
    g2 = gamma.reshape(1, inp_dim)
    bt2 = beta.reshape(1, inp_dim)
    return pl.pallas_call(
        _fused_chunk,
        grid=(n // block,),
        in_specs=[
            pl.BlockSpec((block, d_a), lambda i: (i, 0)),
            pl.BlockSpec((block, inp_dim), lambda i: (i, 0)),
            pl.BlockSpec((d_a, inp_dim), lambda i: (0, 0)),
            pl.BlockSpec((1, inp_dim), lambda i: (0, 0)),
            pl.BlockSpec((1, inp_dim), lambda i: (0, 0)),
            pl.BlockSpec((1, inp_dim), lambda i: (0, 0)),
        ],
        out_specs=pl.BlockSpec((block, inp_dim), lambda i: (i, 0)),
        out_shape=jax.ShapeDtypeStruct((n, inp_dim), jnp.float32),
    )(a, priors, wt, b2, g2, bt2)


# speedup vs baseline: 1.0868x; 1.0284x over previous
import jax
import jax.numpy as jnp
from jax.experimental import pallas as pl
from jax.experimental.pallas import tpu as pltpu

_VBS = 1024
_EPS = 1e-5
_N_BISECT = 10
_N_NEWTON = 2


def _fused_chunk(a_ref, p_ref, wt_ref, b_ref, g_ref, bt_ref, o_ref):
    d = jnp.float32(o_ref.shape[1])
    h = jnp.dot(a_ref[...], wt_ref[...], preferred_element_type=jnp.float32)
    h = h + b_ref[...]
    outs = []
    for j in range(0, h.shape[0], _VBS):
        hc = h[j:j + _VBS]
        mean = jnp.mean(hc, axis=0, keepdims=True)
        var = jnp.mean(jnp.square(hc - mean), axis=0, keepdims=True)
        outs.append((hc - mean) * jax.lax.rsqrt(var + _EPS))
    hn = outs[0] if len(outs) == 1 else jnp.concatenate(outs, axis=0)
    z = (hn * g_ref[...] + bt_ref[...]) * p_ref[...]
    zt = z.T
    m = jnp.max(zt, axis=0, keepdims=True)
    lo = m - 1.0
    for i in range(1, _N_BISECT + 1):
        cand = lo + jnp.float32(2.0 ** -i)
        fs = jnp.sum(jnp.maximum(zt, cand), axis=0, keepdims=True)
        lo = jnp.where(fs - d * cand > 1.0, cand, lo)
    tau = lo
    for _ in range(_N_NEWTON):
        fs = jnp.sum(jnp.maximum(zt, tau), axis=0, keepdims=True)
        k = jnp.sum((zt > tau).astype(jnp.float32), axis=0, keepdims=True)
        tau = tau + (fs - d * tau - 1.0) / k
    o_ref[...] = jnp.maximum(z - tau.T, 0.0)


def kernel(a, priors, W, b, gamma, beta):
    n, d_a = a.shape
    inp_dim = W.shape[0]
    block = min(n, 4 * _VBS)
    wt = W.T
    b2 = b.reshape(1, inp_dim)
    g2 = gamma.reshape(1, inp_dim)
    bt2 = beta.reshape(1, inp_dim)
    return pl.pallas_call(
        _fused_chunk,
        grid=(n // block,),
        in_specs=[
            pl.BlockSpec((block, d_a), lambda i: (i, 0)),
            pl.BlockSpec((block, inp_dim), lambda i: (i, 0)),
            pl.BlockSpec((d_a, inp_dim), lambda i: (0, 0)),
            pl.BlockSpec((1, inp_dim), lambda i: (0, 0)),
            pl.BlockSpec((1, inp_dim), lambda i: (0, 0)),
            pl.BlockSpec((1, inp_dim), lambda i: (0, 0)),
        ],
        out_specs=pl.BlockSpec((block, inp_dim), lambda i: (i, 0)),
        out_shape=jax.ShapeDtypeStruct((n, inp_dim), jnp.float32),
    )(a, priors, wt, b2, g2, bt2)


# speedup vs baseline: 1.1132x; 1.0242x over previous
import jax
import jax.numpy as jnp
from jax.experimental import pallas as pl
from jax.experimental.pallas import tpu as pltpu

_VBS = 1024
_EPS = 1e-5
_N_BISECT = 9
_N_NEWTON = 2


def _fused_chunk(a_ref, p_ref, wt_ref, b_ref, g_ref, bt_ref, o_ref):
    d = jnp.float32(o_ref.shape[1])
    h = jnp.dot(a_ref[...], wt_ref[...], preferred_element_type=jnp.float32)
    h = h + b_ref[...]
    outs = []
    for j in range(0, h.shape[0], _VBS):
        hc = h[j:j + _VBS]
        mean = jnp.mean(hc, axis=0, keepdims=True)
        var = jnp.mean(jnp.square(hc - mean), axis=0, keepdims=True)
        outs.append((hc - mean) * jax.lax.rsqrt(var + _EPS))
    hn = outs[0] if len(outs) == 1 else jnp.concatenate(outs, axis=0)
    z = (hn * g_ref[...] + bt_ref[...]) * p_ref[...]
    zt = z.T
    m = jnp.max(zt, axis=0, keepdims=True)
    lo = m - 1.0
    for i in range(1, _N_BISECT + 1):
        cand = lo + jnp.float32(2.0 ** -i)
        fs = jnp.sum(jnp.maximum(zt, cand), axis=0, keepdims=True)
        lo = jnp.where(fs - d * cand > 1.0, cand, lo)
    tau = lo
    for _ in range(_N_NEWTON):
        fs = jnp.sum(jnp.maximum(zt, tau), axis=0, keepdims=True)
        k = jnp.sum((zt > tau).astype(jnp.float32), axis=0, keepdims=True)
        tau = tau + (fs - d * tau - 1.0) / k
    o_ref[...] = jnp.maximum(z - tau.T, 0.0)


def kernel(a, priors, W, b, gamma, beta):
    n, d_a = a.shape
    inp_dim = W.shape[0]
    block = min(n, 4 * _VBS)
    wt = W.T
    b2 = b.reshape(1, inp_dim)
    g2 = gamma.reshape(1, inp_dim)
    bt2 = beta.reshape(1, inp_dim)
    return pl.pallas_call(
        _fused_chunk,
        grid=(n // block,),
        in_specs=[
            pl.BlockSpec((block, d_a), lambda i: (i, 0)),
            pl.BlockSpec((block, inp_dim), lambda i: (i, 0)),
            pl.BlockSpec((d_a, inp_dim), lambda i: (0, 0)),
            pl.BlockSpec((1, inp_dim), lambda i: (0, 0)),
            pl.BlockSpec((1, inp_dim), lambda i: (0, 0)),
            pl.BlockSpec((1, inp_dim), lambda i: (0, 0)),
        ],
        out_specs=pl.BlockSpec((block, inp_dim), lambda i: (i, 0)),
        out_shape=jax.ShapeDtypeStruct((n, inp_dim), jnp.float32),
    )(a, priors, wt, b2, g2, bt2)


# speedup vs baseline: 1.1411x; 1.0251x over previous
import jax
import jax.numpy as jnp
from jax.experimental import pallas as pl
from jax.experimental.pallas import tpu as pltpu

_VBS = 1024
_EPS = 1e-5
_N_BISECT = 8
_N_NEWTON = 2


def _fused_chunk(a_ref, p_ref, wt_ref, b_ref, g_ref, bt_ref, o_ref):
    d = jnp.float32(o_ref.shape[1])
    h = jnp.dot(a_ref[...], wt_ref[...], preferred_element_type=jnp.float32)
    h = h + b_ref[...]
    outs = []
    for j in range(0, h.shape[0], _VBS):
        hc = h[j:j + _VBS]
        mean = jnp.mean(hc, axis=0, keepdims=True)
        var = jnp.mean(jnp.square(hc - mean), axis=0, keepdims=True)
        outs.append((hc - mean) * jax.lax.rsqrt(var + _EPS))
    hn = outs[0] if len(outs) == 1 else jnp.concatenate(outs, axis=0)
    z = (hn * g_ref[...] + bt_ref[...]) * p_ref[...]
    zt = z.T
    m = jnp.max(zt, axis=0, keepdims=True)
    lo = m - 1.0
    for i in range(1, _N_BISECT + 1):
        cand = lo + jnp.float32(2.0 ** -i)
        fs = jnp.sum(jnp.maximum(zt, cand), axis=0, keepdims=True)
        lo = jnp.where(fs - d * cand > 1.0, cand, lo)
    tau = lo
    for _ in range(_N_NEWTON):
        fs = jnp.sum(jnp.maximum(zt, tau), axis=0, keepdims=True)
        k = jnp.sum((zt > tau).astype(jnp.float32), axis=0, keepdims=True)
        tau = tau + (fs - d * tau - 1.0) / k
    o_ref[...] = jnp.maximum(z - tau.T, 0.0)


def kernel(a, priors, W, b, gamma, beta):
    n, d_a = a.shape
    inp_dim = W.shape[0]
    block = min(n, 4 * _VBS)
    wt = W.T
    b2 = b.reshape(1, inp_dim)
    g2 = gamma.reshape(1, inp_dim)
    bt2 = beta.reshape(1, inp_dim)
    return pl.pallas_call(
        _fused_chunk,
        grid=(n // block,),
        in_specs=[
            pl.BlockSpec((block, d_a), lambda i: (i, 0)),
            pl.BlockSpec((block, inp_dim), lambda i: (i, 0)),
            pl.BlockSpec((d_a, inp_dim), lambda i: (0, 0)),
            pl.BlockSpec((1, inp_dim), lambda i: (0, 0)),
            pl.BlockSpec((1, inp_dim), lambda i: (0, 0)),
            pl.BlockSpec((1, inp_dim), lambda i: (0, 0)),
        ],
        out_specs=pl.BlockSpec((block, inp_dim), lambda i: (i, 0)),
        out_shape=jax.ShapeDtypeStruct((n, inp_dim), jnp.float32),
    )(a, priors, wt, b2, g2, bt2)
